# baseline (speedup 1.0000x reference)
import functools
import jax
import jax.numpy as jnp
from jax import lax
from jax.experimental import pallas as pl
from jax.experimental.pallas import tpu as pltpu
from jax.experimental.pallas import tpu_sc as plsc

N = 200000
NB = 30
NBP = 32
NC = 16


_NW = 32
_NT = 16
_CH = 2048
_NFULL = N // _CH
_TAIL = N - _NFULL * _CH
_HSZ = 2 * NC * NBP * NBP
_HSTR = NBP * NBP + 1

_sc_mesh = plsc.VectorSubcoreMesh(core_axis_name="c", subcore_axis_name="s")


@functools.partial(
    pl.kernel, mesh=_sc_mesh,
    compiler_params=pltpu.CompilerParams(needs_layout_passes=False,
                                         use_tc_tiling_on_sc=False),
    out_type=[jax.ShapeDtypeStruct((_NW, _HSZ), jnp.float32),
              jax.ShapeDtypeStruct((2, 128), jnp.float32)],
    scratch_types=[
        pltpu.VMEM((_CH,), jnp.float32),
        pltpu.VMEM((_CH,), jnp.float32),
        pltpu.VMEM((_CH, NC), jnp.float32),
        pltpu.VMEM((_CH, NC), jnp.float32),
        pltpu.VMEM((2 * NC * _HSTR,), jnp.float32),
        pltpu.VMEM((128,), jnp.float32),
        pltpu.VMEM((_NT, 128), jnp.float32),
        pltpu.VMEM_SHARED((_NT, 128), jnp.float32),
        pltpu.SemaphoreType.DMA,
    ],
)
def _sc_main(yt_hbm, yp_hbm, u_hbm, s_hbm, hist_hbm, bnd_hbm,
             yt_v, yp_v, u_v, s_v, h_v, pack_v, fold_v, shared_v, dsem):
    cid = lax.axis_index("c")
    sid = lax.axis_index("s")
    wid = sid * 2 + cid

    inf = jnp.float32(jnp.inf)
    big = jnp.full((NC,), inf, jnp.float32)
    zero16 = jnp.zeros((NC,), jnp.float32)

    def load_chunk(g, nvec, with_yt):
        base = g * _CH
        ne = nvec * 16
        cps = []
        if with_yt:
            cps.append(pltpu.async_copy(yt_hbm.at[pl.ds(base, ne)],
                                        yt_v.at[pl.ds(0, ne)], dsem))
        cps.append(pltpu.async_copy(yp_hbm.at[pl.ds(base, ne)],
                                    yp_v.at[pl.ds(0, ne)], dsem))
        cps.append(pltpu.async_copy(u_hbm.at[pl.ds(base, ne)],
                                    u_v.at[pl.ds(0, ne)], dsem))
        cps.append(pltpu.async_copy(s_hbm.at[pl.ds(base, ne)],
                                    s_v.at[pl.ds(0, ne)], dsem))
        for cp in cps:
            cp.wait()

    def bounds_chunk(g, nvec, acc):
        load_chunk(g, nvec, True)
        umin, umax, smin, smax, ypmin, ypmax, sab = acc

        def ybody(v, yacc):
            ymn, ymx, sb = yacc
            yt16 = yt_v[pl.ds(v * 16, 16)]
            yp16 = yp_v[pl.ds(v * 16, 16)]
            return (jnp.minimum(ymn, yp16), jnp.maximum(ymx, yp16),
                    sb + jnp.abs(yt16 - yp16))
        ypmin, ypmax, sab = lax.fori_loop(0, nvec, ybody,
                                          (ypmin, ypmax, sab))

        def rbody(e, racc):
            umn, umx, smn, smx = racc
            ur = u_v[e]
            sr = s_v[e]
            return (jnp.minimum(umn, ur), jnp.maximum(umx, ur),
                    jnp.minimum(smn, sr), jnp.maximum(smx, sr))
        umin, umax, smin, smax = lax.fori_loop(0, nvec * 16, rbody,
                                               (umin, umax, smin, smax))
        return (umin, umax, smin, smax, ypmin, ypmax, sab)

    acc0 = (big, -big, big, -big, big, -big, zero16)
    nfull_a = jnp.where(sid < _NFULL % _NT, (_NFULL // _NT) + 1,
                        _NFULL // _NT)

    def abody(k, acc):
        return bounds_chunk(sid + _NT * k, _CH // 16, acc)
    acc = lax.fori_loop(0, nfull_a, abody, acc0)
    acc = lax.cond(sid == _NFULL % _NT,
                   lambda a: bounds_chunk(jnp.int32(_NFULL), _TAIL // 16, a),
                   lambda a: a, acc)
    umin, umax, smin, smax, ypmin, ypmax, sab = acc

    pack_v[pl.ds(0, 16)] = umin
    pack_v[pl.ds(16, 16)] = umax
    pack_v[pl.ds(32, 16)] = smin
    pack_v[pl.ds(48, 16)] = smax
    pack_v[pl.ds(64, 16)] = ypmin
    pack_v[pl.ds(80, 16)] = ypmax
    pack_v[pl.ds(96, 16)] = sab
    pack_v[pl.ds(112, 16)] = zero16
    pltpu.sync_copy(pack_v, shared_v.at[sid])
    plsc.subcore_barrier()
    pltpu.sync_copy(shared_v, fold_v)

    umin, umax, smin, smax = big, -big, big, -big
    ypmin, ypmax, sab = big, -big, zero16
    for t in range(_NT):
        umin = jnp.minimum(umin, fold_v[t, pl.ds(0, 16)])
        umax = jnp.maximum(umax, fold_v[t, pl.ds(16, 16)])
        smin = jnp.minimum(smin, fold_v[t, pl.ds(32, 16)])
        smax = jnp.maximum(smax, fold_v[t, pl.ds(48, 16)])
        ypmin = jnp.minimum(ypmin, fold_v[t, pl.ds(64, 16)])
        ypmax = jnp.maximum(ypmax, fold_v[t, pl.ds(80, 16)])
        sab = sab + fold_v[t, pl.ds(96, 16)]

    nbf = jnp.full((NC,), jnp.float32(NB), jnp.float32)
    uscale = nbf / jnp.maximum(umax - umin, 1e-12)
    sscale = nbf / jnp.maximum(smax - smin, 1e-12)
    ymin = jnp.min(ypmin)
    ymax = jnp.max(ypmax)
    yr = jnp.maximum(ymax - ymin, 1e-12)
    yscale = (nbf / jnp.full((NC,), yr, jnp.float32))[0]

    @pl.when(sid == 0)
    def _emit_bounds():
        pack_v[pl.ds(0, 16)] = sab
        pltpu.sync_copy(pack_v, bnd_hbm.at[cid])

    iota16 = lax.broadcasted_iota(jnp.int32, (NC,), 0)
    coloff_u = iota16 * _HSTR
    coloff_s = coloff_u + NC * _HSTR
    ones = jnp.full((NC,), 1.0, jnp.float32)

    def zbody(i, _):
        h_v[pl.ds(i * 16, 16)] = jnp.zeros((16,), jnp.float32)
        return 0
    lax.fori_loop(0, (2 * NC * _HSTR) // 16, zbody, 0)

    def process(g, nvec):
        load_chunk(g, nvec, False)

        @functools.partial(plsc.parallel_loop, 0, nvec, unroll=2)
        def ebody(v):
            yv = yp_v[pl.ds(v * 16, 16)]
            yiv = jnp.minimum(((yv - ymin) * yscale).astype(jnp.int32), NB - 1)
            for j in range(16):
                yi_e = yiv[j]
                e = v * 16 + j
                urow = u_v[e]
                xiu = jnp.minimum(((urow - umin) * uscale).astype(jnp.int32),
                                  NB - 1)
                plsc.addupdate_scatter(h_v, [coloff_u + xiu * NBP + yi_e], ones)
                srow = s_v[e]
                xis = jnp.minimum(((srow - smin) * sscale).astype(jnp.int32),
                                  NB - 1)
                plsc.addupdate_scatter(h_v, [coloff_s + xis * NBP + yi_e], ones)

    nfull_b = jnp.where(wid < _NFULL % _NW, (_NFULL // _NW) + 1,
                        _NFULL // _NW)

    def cbody(k, _):
        process(wid + _NW * k, _CH // 16)
        return 0
    lax.fori_loop(0, nfull_b, cbody, 0)

    @pl.when(wid == _NFULL % _NW)
    def _tail():
        process(jnp.int32(_NFULL), _TAIL // 16)

    for cs in range(1, 2 * NC):
        def mvbody(i, _):
            h_v[pl.ds(cs * NBP * NBP + i * 16, 16)] = (
                h_v[pl.ds(cs * _HSTR + i * 16, 16)])
            return 0
        lax.fori_loop(0, (NBP * NBP) // 16, mvbody, 0)

    pltpu.sync_copy(h_v.at[pl.ds(0, _HSZ)], hist_hbm.at[wid])


def _stage12(y_true, y_pred, u_vec, s_vec):
    parts, bnd = _sc_main(y_true, y_pred, u_vec, s_vec)
    return parts.reshape(_NW, 2 * NC * NBP, NBP), bnd


LOSS_L = 0.1
LOSS_A = 0.5


def _finalize_body(hist_ref, bnd_ref, uh_ref, sh_ref, out_ref):
    h = jnp.sum(hist_ref[...], axis=0).reshape(2, NC, NBP, NBP)[:, :, :NB, :NB]
    nx = jnp.sum(h, axis=3)
    ny = jnp.sum(h, axis=2)
    sx = jnp.sum(nx, axis=2, keepdims=True)
    sy = jnp.sum(ny, axis=2, keepdims=True)
    sxy = jnp.sum(h, axis=(2, 3), keepdims=True)
    px = nx / sx
    py = ny / sy
    pxy = h / sxy
    px = jnp.where(px == 0, 1e-10, px)
    py = jnp.where(py == 0, 1e-10, py)
    pxy = jnp.where(pxy == 0, 1e-10, pxy)
    outer = px[:, :, :, None] * py[:, :, None, :]
    mi = jnp.sum(pxy * jnp.log(pxy / outer), axis=(2, 3))
    mi_uq = jnp.sum(mi[0])
    mi_sq = jnp.sum(mi[1])

    t = (lax.broadcasted_iota(jnp.int32, (1, NB), 1).astype(jnp.float32) /
         jnp.float32(NB - 1))

    def digitize(vm):
        mn = jnp.min(vm, axis=1, keepdims=True)
        mx = jnp.max(vm, axis=1, keepdims=True)
        edges = mn * (1.0 - t) + mx * t
        return jnp.sum((edges[:, None, :] <= vm[:, :, None]).astype(jnp.int32),
                       axis=2)

    la = digitize(uh_ref[...])
    lb = digitize(sh_ref[...])
    eq_a = (la[:, :, None] == la[:, None, :])
    eq_b = (lb[:, :, None] == lb[:, None, :])
    cx = jnp.sum(eq_a.astype(jnp.float32), axis=2)
    cy = jnp.sum(eq_b.astype(jnp.float32), axis=2)
    cxy = jnp.sum((eq_a & eq_b).astype(jnp.float32), axis=2)
    n16 = jnp.float32(NC)
    mi_us = jnp.sum(jnp.log(cxy * n16 / (cx * cy))) / n16

    l_pt = jnp.sum(bnd_ref[0, 0:16]) / jnp.float32(N)
    out_ref[...] = jnp.full((1, 1), l_pt + LOSS_L * (mi_uq + mi_sq - LOSS_A * mi_us),
                            jnp.float32)


def _finalize(hist, bnd, u_head, s_head):
    return pl.pallas_call(
        _finalize_body,
        out_shape=jax.ShapeDtypeStruct((1, 1), jnp.float32),
    )(hist, bnd, u_head, s_head)


@jax.jit
def kernel(y_true, y_pred, u_attr, s_attr, u_vec, s_vec):
    hist, bnd = _stage12(y_true, y_pred, u_vec, s_vec)
    out = _finalize(hist, bnd, u_vec[:NC, :], s_vec[:NC, :])
    return out[0, 0]


# speedup vs baseline: 1.0863x; 1.0863x over previous
import functools
import jax
import jax.numpy as jnp
from jax import lax
from jax.experimental import pallas as pl
from jax.experimental.pallas import tpu as pltpu
from jax.experimental.pallas import tpu_sc as plsc

N = 200000
NB = 30
NBP = 32
NC = 16


_NW = 32
_NT = 16
_CH = 2048
_NFULL = N // _CH
_TAIL = N - _NFULL * _CH
_HSZ = 2 * NC * NBP * NBP
_HSTR = NBP * NBP + 1

_sc_mesh = plsc.VectorSubcoreMesh(core_axis_name="c", subcore_axis_name="s")


@functools.partial(
    pl.kernel, mesh=_sc_mesh,
    compiler_params=pltpu.CompilerParams(needs_layout_passes=False,
                                         use_tc_tiling_on_sc=False),
    out_type=[jax.ShapeDtypeStruct((_NW, _HSZ), jnp.float32),
              jax.ShapeDtypeStruct((2, 128), jnp.float32)],
    scratch_types=[
        pltpu.VMEM((_CH,), jnp.float32),
        pltpu.VMEM((_CH,), jnp.float32),
        pltpu.VMEM((_CH, NC), jnp.float32),
        pltpu.VMEM((_CH, NC), jnp.float32),
        pltpu.VMEM((2 * NC * _HSTR,), jnp.float32),
        pltpu.VMEM((128,), jnp.float32),
        pltpu.VMEM((_NT, 128), jnp.float32),
        pltpu.VMEM_SHARED((_NT, 128), jnp.float32),
        pltpu.SemaphoreType.DMA,
    ],
)
def _sc_main(yt_hbm, yp_hbm, u_hbm, s_hbm, hist_hbm, bnd_hbm,
             yt_v, yp_v, u_v, s_v, h_v, pack_v, fold_v, shared_v, dsem):
    cid = lax.axis_index("c")
    sid = lax.axis_index("s")
    wid = sid * 2 + cid

    inf = jnp.float32(jnp.inf)
    big = jnp.full((NC,), inf, jnp.float32)
    zero16 = jnp.zeros((NC,), jnp.float32)

    def load_chunk(g, nvec, with_yt):
        base = g * _CH
        ne = nvec * 16
        cps = []
        if with_yt:
            cps.append(pltpu.async_copy(yt_hbm.at[pl.ds(base, ne)],
                                        yt_v.at[pl.ds(0, ne)], dsem))
        cps.append(pltpu.async_copy(yp_hbm.at[pl.ds(base, ne)],
                                    yp_v.at[pl.ds(0, ne)], dsem))
        cps.append(pltpu.async_copy(u_hbm.at[pl.ds(base, ne)],
                                    u_v.at[pl.ds(0, ne)], dsem))
        cps.append(pltpu.async_copy(s_hbm.at[pl.ds(base, ne)],
                                    s_v.at[pl.ds(0, ne)], dsem))
        for cp in cps:
            cp.wait()

    def bounds_chunk(g, nvec, acc):
        load_chunk(g, nvec, True)
        umin, umax, smin, smax, ypmin, ypmax, sab = acc

        def ybody(v, yacc):
            ymn, ymx, sb = yacc
            yt16 = yt_v[pl.ds(v * 16, 16)]
            yp16 = yp_v[pl.ds(v * 16, 16)]
            return (jnp.minimum(ymn, yp16), jnp.maximum(ymx, yp16),
                    sb + jnp.abs(yt16 - yp16))
        ypmin, ypmax, sab = lax.fori_loop(0, nvec, ybody,
                                          (ypmin, ypmax, sab))

        def rbody(q, racc):
            a0, b0, c0, d0, a1, b1, c1, d1 = racc
            u0 = u_v[2 * q]
            s0 = s_v[2 * q]
            u1 = u_v[2 * q + 1]
            s1 = s_v[2 * q + 1]
            return (jnp.minimum(a0, u0), jnp.maximum(b0, u0),
                    jnp.minimum(c0, s0), jnp.maximum(d0, s0),
                    jnp.minimum(a1, u1), jnp.maximum(b1, u1),
                    jnp.minimum(c1, s1), jnp.maximum(d1, s1))
        r = lax.fori_loop(0, nvec * 8, rbody,
                          (umin, umax, smin, smax, big, -big, big, -big))
        umin = jnp.minimum(r[0], r[4])
        umax = jnp.maximum(r[1], r[5])
        smin = jnp.minimum(r[2], r[6])
        smax = jnp.maximum(r[3], r[7])
        return (umin, umax, smin, smax, ypmin, ypmax, sab)

    acc0 = (big, -big, big, -big, big, -big, zero16)
    nfull_a = jnp.where(sid < _NFULL % _NT, (_NFULL // _NT) + 1,
                        _NFULL // _NT)

    def abody(k, acc):
        return bounds_chunk(sid + _NT * k, _CH // 16, acc)
    acc = lax.fori_loop(0, nfull_a, abody, acc0)
    acc = lax.cond(sid == _NFULL % _NT,
                   lambda a: bounds_chunk(jnp.int32(_NFULL), _TAIL // 16, a),
                   lambda a: a, acc)
    umin, umax, smin, smax, ypmin, ypmax, sab = acc

    pack_v[pl.ds(0, 16)] = umin
    pack_v[pl.ds(16, 16)] = umax
    pack_v[pl.ds(32, 16)] = smin
    pack_v[pl.ds(48, 16)] = smax
    pack_v[pl.ds(64, 16)] = ypmin
    pack_v[pl.ds(80, 16)] = ypmax
    pack_v[pl.ds(96, 16)] = sab
    pack_v[pl.ds(112, 16)] = zero16
    pltpu.sync_copy(pack_v, shared_v.at[sid])
    plsc.subcore_barrier()
    pltpu.sync_copy(shared_v, fold_v)

    umin, umax, smin, smax = big, -big, big, -big
    ypmin, ypmax, sab = big, -big, zero16
    for t in range(_NT):
        umin = jnp.minimum(umin, fold_v[t, pl.ds(0, 16)])
        umax = jnp.maximum(umax, fold_v[t, pl.ds(16, 16)])
        smin = jnp.minimum(smin, fold_v[t, pl.ds(32, 16)])
        smax = jnp.maximum(smax, fold_v[t, pl.ds(48, 16)])
        ypmin = jnp.minimum(ypmin, fold_v[t, pl.ds(64, 16)])
        ypmax = jnp.maximum(ypmax, fold_v[t, pl.ds(80, 16)])
        sab = sab + fold_v[t, pl.ds(96, 16)]

    nbf = jnp.full((NC,), jnp.float32(NB), jnp.float32)
    uscale = nbf / jnp.maximum(umax - umin, 1e-12)
    sscale = nbf / jnp.maximum(smax - smin, 1e-12)
    ymin = jnp.min(ypmin)
    ymax = jnp.max(ypmax)
    yr = jnp.maximum(ymax - ymin, 1e-12)
    yscale = (nbf / jnp.full((NC,), yr, jnp.float32))[0]

    @pl.when(sid == 0)
    def _emit_bounds():
        pack_v[pl.ds(0, 16)] = sab
        pltpu.sync_copy(pack_v, bnd_hbm.at[cid])

    iota16 = lax.broadcasted_iota(jnp.int32, (NC,), 0)
    coloff_u = iota16 * _HSTR
    coloff_s = coloff_u + NC * _HSTR
    ones = jnp.full((NC,), 1.0, jnp.float32)

    def zbody(i, _):
        h_v[pl.ds(i * 16, 16)] = jnp.zeros((16,), jnp.float32)
        return 0
    lax.fori_loop(0, (2 * NC * _HSTR) // 16, zbody, 0)

    def process(g, nvec):
        load_chunk(g, nvec, False)

        @functools.partial(plsc.parallel_loop, 0, nvec, unroll=2)
        def ebody(v):
            yv = yp_v[pl.ds(v * 16, 16)]
            yiv = jnp.minimum(((yv - ymin) * yscale).astype(jnp.int32), NB - 1)
            for j in range(16):
                yi_e = yiv[j]
                e = v * 16 + j
                urow = u_v[e]
                xiu = jnp.minimum(((urow - umin) * uscale).astype(jnp.int32),
                                  NB - 1)
                plsc.addupdate_scatter(h_v, [coloff_u + xiu * NBP + yi_e], ones)
                srow = s_v[e]
                xis = jnp.minimum(((srow - smin) * sscale).astype(jnp.int32),
                                  NB - 1)
                plsc.addupdate_scatter(h_v, [coloff_s + xis * NBP + yi_e], ones)

    nfull_b = jnp.where(wid < _NFULL % _NW, (_NFULL // _NW) + 1,
                        _NFULL // _NW)

    def cbody(k, _):
        process(wid + _NW * k, _CH // 16)
        return 0
    lax.fori_loop(0, nfull_b, cbody, 0)

    @pl.when(wid == _NFULL % _NW)
    def _tail():
        process(jnp.int32(_NFULL), _TAIL // 16)

    for cs in range(1, 2 * NC):
        def mvbody(i, _):
            h_v[pl.ds(cs * NBP * NBP + i * 16, 16)] = (
                h_v[pl.ds(cs * _HSTR + i * 16, 16)])
            return 0
        lax.fori_loop(0, (NBP * NBP) // 16, mvbody, 0)

    pltpu.sync_copy(h_v.at[pl.ds(0, _HSZ)], hist_hbm.at[wid])


def _stage12(y_true, y_pred, u_vec, s_vec):
    parts, bnd = _sc_main(y_true, y_pred, u_vec, s_vec)
    return parts.reshape(_NW, 2 * NC * NBP, NBP), bnd


LOSS_L = 0.1
LOSS_A = 0.5


def _finalize_body(hist_ref, bnd_ref, uh_ref, sh_ref, out_ref):
    h = jnp.sum(hist_ref[...], axis=0).reshape(2, NC, NBP, NBP)[:, :, :NB, :NB]
    nx = jnp.sum(h, axis=3)
    ny = jnp.sum(h, axis=2)
    sx = jnp.sum(nx, axis=2, keepdims=True)
    sy = jnp.sum(ny, axis=2, keepdims=True)
    sxy = jnp.sum(h, axis=(2, 3), keepdims=True)
    px = nx / sx
    py = ny / sy
    pxy = h / sxy
    px = jnp.where(px == 0, 1e-10, px)
    py = jnp.where(py == 0, 1e-10, py)
    pxy = jnp.where(pxy == 0, 1e-10, pxy)
    outer = px[:, :, :, None] * py[:, :, None, :]
    mi = jnp.sum(pxy * jnp.log(pxy / outer), axis=(2, 3))
    mi_uq = jnp.sum(mi[0])
    mi_sq = jnp.sum(mi[1])

    t = (lax.broadcasted_iota(jnp.int32, (1, NB), 1).astype(jnp.float32) /
         jnp.float32(NB - 1))

    def digitize(vm):
        mn = jnp.min(vm, axis=1, keepdims=True)
        mx = jnp.max(vm, axis=1, keepdims=True)
        edges = mn * (1.0 - t) + mx * t
        return jnp.sum((edges[:, None, :] <= vm[:, :, None]).astype(jnp.int32),
                       axis=2)

    la = digitize(uh_ref[...])
    lb = digitize(sh_ref[...])
    eq_a = (la[:, :, None] == la[:, None, :])
    eq_b = (lb[:, :, None] == lb[:, None, :])
    cx = jnp.sum(eq_a.astype(jnp.float32), axis=2)
    cy = jnp.sum(eq_b.astype(jnp.float32), axis=2)
    cxy = jnp.sum((eq_a & eq_b).astype(jnp.float32), axis=2)
    n16 = jnp.float32(NC)
    mi_us = jnp.sum(jnp.log(cxy * n16 / (cx * cy))) / n16

    l_pt = jnp.sum(bnd_ref[0, 0:16]) / jnp.float32(N)
    out_ref[...] = jnp.full((1, 1), l_pt + LOSS_L * (mi_uq + mi_sq - LOSS_A * mi_us),
                            jnp.float32)


def _finalize(hist, bnd, u_head, s_head):
    return pl.pallas_call(
        _finalize_body,
        out_shape=jax.ShapeDtypeStruct((1, 1), jnp.float32),
    )(hist, bnd, u_head, s_head)


@jax.jit
def kernel(y_true, y_pred, u_attr, s_attr, u_vec, s_vec):
    hist, bnd = _stage12(y_true, y_pred, u_vec, s_vec)
    out = _finalize(hist, bnd, u_vec[:NC, :], s_vec[:NC, :])
    return out[0, 0]
